# baseline (speedup 1.0000x reference)
import functools

import jax
import jax.numpy as jnp
from jax import lax
from jax.experimental import pallas as pl
from jax.experimental.pallas import tpu as pltpu
from jax.experimental.pallas import tpu_sc as plsc

VOCAB_SIZE = 1000000
EMBED_DIM = 32
BATCH = 4096
SEQ_LEN = 200

_info = plsc.get_sparse_core_info()
_NC, _NS = _info.num_cores, _info.num_subcores
_NW = _NC * _NS
_SG = SEQ_LEN // 8
_BB = BATCH // 128
_CHUNK = 256
_NCHUNK = (_SG * 1024) // _CHUNK


@functools.partial(
    pl.kernel,
    mesh=plsc.VectorSubcoreMesh(core_axis_name="c", subcore_axis_name="s"),
    out_type=jax.ShapeDtypeStruct((SEQ_LEN, 4, _BB, 8, 128), jnp.float32),
    scratch_types=[
        pltpu.VMEM((_CHUNK,), jnp.int32),
        pltpu.VMEM((_CHUNK,), jnp.int32),
        pltpu.VMEM((_CHUNK,), jnp.int32),
        pltpu.VMEM((_CHUNK,), jnp.int32),
        pltpu.VMEM((_CHUNK, 128), jnp.float32),
        pltpu.VMEM((_CHUNK, 128), jnp.float32),
        pltpu.VMEM((32, 129), jnp.float32),
        pltpu.VMEM((32, 129), jnp.float32),
        pltpu.SemaphoreType.DMA,
        pltpu.SemaphoreType.DMA,
        pltpu.SemaphoreType.DMA,
        pltpu.SemaphoreType.DMA,
    ],
    compiler_params=pltpu.CompilerParams(
        use_tc_tiling_on_sc=True, needs_layout_passes=False
    ),
)
def _sc_gather(
    t128_hbm, x4_hbm, out_hbm,
    idx_a, idx_b, lin_a, lin_b, rows_a, rows_b, tbuf_a, tbuf_b,
    gsem_a, gsem_b, wsem_a, wsem_b,
):
    wid = lax.axis_index("s") * _NC + lax.axis_index("c")
    iota = lax.iota(jnp.int32, 16)
    iota16p = iota + 16

    def drain_tbuf(tbuf, wsem):
        for t in range(4):
            pltpu.make_async_copy(
                out_hbm.at[0, 0, 0],
                tbuf.at[pl.ds(t * 8, 8), pl.ds(0, 128)],
                wsem,
            ).wait()

    def stage_chunk(c, idx, lin):
        g = c // 4
        q0 = (c % 4) * _CHUNK
        pltpu.sync_copy(x4_hbm.at[g, wid, pl.ds(q0, _CHUNK)], idx)

        def sbody(u, carry):
            v = idx[pl.ds(u * 16, 16)]
            lin[pl.ds(u * 16, 16)] = v >> 2
            return carry

        lax.fori_loop(0, _CHUNK // 16, sbody, 0)

    def transpose_chunk(c, idx, rows, first):
        def do_sh(sh, tbuf, wsem, guard):
            if guard is None:
                drain_tbuf(tbuf, wsem)
            else:

                @pl.when(guard)
                def _():
                    drain_tbuf(tbuf, wsem)

            row0 = sh * 128

            def ubody(u, carry):
                bl0 = u * 16
                qv = (idx[pl.ds(row0 + bl0, 16)] & 3) << 5
                for k in range(16):
                    bl = bl0 + k
                    r = row0 + bl
                    col0 = qv[k]
                    colv = jnp.full((16,), bl, jnp.int32)
                    h0 = rows[r, pl.ds(col0, 16)]
                    h1 = rows[r, pl.ds(col0 + 16, 16)]
                    plsc.store_scatter(tbuf, [iota, colv], h0)
                    plsc.store_scatter(tbuf, [iota16p, colv], h1)
                return carry

            lax.fori_loop(0, 8, ubody, 0)
            s = c * 2 + sh
            for t in range(4):
                pltpu.async_copy(
                    tbuf.at[pl.ds(t * 8, 8), pl.ds(0, 128)],
                    out_hbm.at[s, t, wid],
                    wsem,
                )

        do_sh(0, tbuf_a, wsem_a, (c > 0) if first else None)
        do_sh(1, tbuf_b, wsem_b, (c > 0) if first else None)

    stage_chunk(0, idx_a, lin_a)
    pltpu.async_copy(t128_hbm.at[lin_a], rows_a, gsem_a)

    def kbody(k, carry):
        c = k * 2
        stage_chunk(c + 1, idx_b, lin_b)
        pltpu.async_copy(t128_hbm.at[lin_b], rows_b, gsem_b)
        pltpu.make_async_copy(t128_hbm.at[lin_a], rows_a, gsem_a).wait()
        transpose_chunk(c, idx_a, rows_a, True)

        @pl.when(c + 2 < _NCHUNK)
        def _():
            stage_chunk(c + 2, idx_a, lin_a)
            pltpu.async_copy(t128_hbm.at[lin_a], rows_a, gsem_a)

        pltpu.make_async_copy(t128_hbm.at[lin_b], rows_b, gsem_b).wait()
        transpose_chunk(c + 1, idx_b, rows_b, False)
        return carry

    lax.fori_loop(0, _NCHUNK // 2, kbody, 0)
    drain_tbuf(tbuf_a, wsem_a)
    drain_tbuf(tbuf_b, wsem_b)


def kernel(x, embedding_weight):
    x4 = (
        x.T.reshape(_SG, 8, _BB, 128).transpose(0, 2, 1, 3).reshape(_SG, _BB, 1024)
    )
    t128 = embedding_weight.reshape(VOCAB_SIZE // 4, 128)
    out6 = _sc_gather(t128, x4)
    return out6.transpose(2, 4, 0, 1, 3).reshape(BATCH, SEQ_LEN, EMBED_DIM)


# speedup vs baseline: 1.6502x; 1.6502x over previous
import functools

import jax
import jax.numpy as jnp
from jax import lax
from jax.experimental import pallas as pl
from jax.experimental.pallas import tpu as pltpu
from jax.experimental.pallas import tpu_sc as plsc

VOCAB_SIZE = 1000000
EMBED_DIM = 32
BATCH = 4096
SEQ_LEN = 200

_info = plsc.get_sparse_core_info()
_NC, _NS = _info.num_cores, _info.num_subcores
_NW = _NC * _NS
_SG = SEQ_LEN // 8
_BB = BATCH // 128
_CHUNK = 1024


@functools.partial(
    pl.kernel,
    mesh=plsc.VectorSubcoreMesh(core_axis_name="c", subcore_axis_name="s"),
    out_type=jax.ShapeDtypeStruct((SEQ_LEN, 4, _BB, 8, 128), jnp.float32),
    scratch_types=[
        pltpu.VMEM((_CHUNK,), jnp.int32),
        pltpu.VMEM((_CHUNK,), jnp.int32),
        pltpu.VMEM((_CHUNK, EMBED_DIM), jnp.float32),
        pltpu.VMEM((_CHUNK, EMBED_DIM), jnp.float32),
        pltpu.VMEM((32, 129), jnp.float32),
        pltpu.VMEM((32, 129), jnp.float32),
        pltpu.SemaphoreType.DMA,
        pltpu.SemaphoreType.DMA,
        pltpu.SemaphoreType.DMA,
        pltpu.SemaphoreType.DMA,
    ],
    compiler_params=pltpu.CompilerParams(
        use_tc_tiling_on_sc=False, needs_layout_passes=False
    ),
)
def _sc_gather(
    table_hbm, x4_hbm, out_hbm,
    idx_a, idx_b, rows_a, rows_b, tbuf_a, tbuf_b,
    gsem_a, gsem_b, wsem_a, wsem_b,
):
    wid = lax.axis_index("s") * _NC + lax.axis_index("c")
    iota = lax.iota(jnp.int32, 16)
    iota16p = iota + 16

    def drain_tbuf(tbuf, wsem):
        for t in range(4):
            pltpu.make_async_copy(
                out_hbm.at[0, 0, 0],
                tbuf.at[pl.ds(t * 8, 8), pl.ds(0, 128)],
                wsem,
            ).wait()

    def transpose_chunk(g, rows, first):
        def do_ss(ss, tbuf, wsem, guard):
            if guard is None:
                drain_tbuf(tbuf, wsem)
            else:

                @pl.when(guard)
                def _():
                    drain_tbuf(tbuf, wsem)

            row0 = ss * 128
            def ubody(u, carry):
                bl0 = u * 4
                for k in range(4):
                    bl = bl0 + k
                    r = row0 + bl
                    colv = jnp.full((16,), bl, jnp.int32)
                    h0 = rows[r, pl.ds(0, 16)]
                    h1 = rows[r, pl.ds(16, 16)]
                    plsc.store_scatter(tbuf, [iota, colv], h0)
                    plsc.store_scatter(tbuf, [iota16p, colv], h1)
                return carry

            lax.fori_loop(0, 32, ubody, 0)
            s = g * 8 + ss
            for t in range(4):
                pltpu.async_copy(
                    tbuf.at[pl.ds(t * 8, 8), pl.ds(0, 128)],
                    out_hbm.at[s, t, wid],
                    wsem,
                )

        def jbody(j, carry):
            ss0 = j * 2
            do_ss(ss0, tbuf_a, wsem_a, (g > 0) | (j > 0) if first else None)
            do_ss(ss0 + 1, tbuf_b, wsem_b, (g > 0) | (j > 0) if first else None)
            return carry

        lax.fori_loop(0, 4, jbody, 0)

    pltpu.sync_copy(x4_hbm.at[0, wid], idx_a)
    ga = pltpu.async_copy(table_hbm.at[idx_a], rows_a, gsem_a)

    def kbody(k, carry):
        g = k * 2
        pltpu.sync_copy(x4_hbm.at[g + 1, wid], idx_b)
        pltpu.async_copy(table_hbm.at[idx_b], rows_b, gsem_b)
        pltpu.make_async_copy(table_hbm.at[idx_a], rows_a, gsem_a).wait()
        transpose_chunk(g, rows_a, True)
        pltpu.sync_copy(x4_hbm.at[g + 2, wid], idx_a)
        pltpu.async_copy(table_hbm.at[idx_a], rows_a, gsem_a)
        pltpu.make_async_copy(table_hbm.at[idx_b], rows_b, gsem_b).wait()
        transpose_chunk(g + 1, rows_b, False)
        return carry

    lax.fori_loop(0, (_SG - 1) // 2, kbody, 0)

    ga = pltpu.make_async_copy(table_hbm.at[idx_a], rows_a, gsem_a)
    ga.wait()
    transpose_chunk(_SG - 1, rows_a, False)
    drain_tbuf(tbuf_a, wsem_a)
    drain_tbuf(tbuf_b, wsem_b)


def kernel(x, embedding_weight):
    x4 = (
        x.T.reshape(_SG, 8, _BB, 128).transpose(0, 2, 1, 3).reshape(_SG, _BB, 1024)
    )
    out6 = _sc_gather(embedding_weight, x4)
    return out6.transpose(2, 4, 0, 1, 3).reshape(BATCH, SEQ_LEN, EMBED_DIM)
